# baseline (speedup 1.0000x reference)
import functools

import jax
import jax.numpy as jnp
from jax import lax
from jax.experimental import pallas as pl
from jax.experimental.pallas import tpu as pltpu
from jax.experimental.pallas import tpu_sc as plsc

D = 64
NT = 7
N_WORKERS = 32
CHUNK = 64


@functools.partial(jax.jit, static_argnums=(1,))
def _sc_embed_call(args, n):
    idx_blk, tables = args[0], args[1:]
    n_per_w = n // N_WORKERS
    n_chunks = n_per_w // CHUNK
    assert n_chunks % 2 == 0 and n_chunks >= 6
    mesh = plsc.VectorSubcoreMesh(core_axis_name="c", subcore_axis_name="s")

    @functools.partial(
        pl.kernel,
        mesh=mesh,
        out_type=jax.ShapeDtypeStruct((n, D), jnp.float32),
        scratch_types=[
            pltpu.VMEM((2, NT * CHUNK), jnp.int32),
            pltpu.VMEM((2, NT, CHUNK), jnp.int32),
            pltpu.VMEM((2, NT, CHUNK, D), jnp.float32),
            pltpu.VMEM((2, CHUNK, D), jnp.float32),
            pltpu.SemaphoreType.DMA,
            pltpu.SemaphoreType.DMA,
            pltpu.SemaphoreType.DMA,
            pltpu.SemaphoreType.DMA,
            pltpu.SemaphoreType.DMA,
            pltpu.SemaphoreType.DMA,
        ],
        compiler_params=pltpu.CompilerParams(use_tc_tiling_on_sc=False,
                                             needs_layout_passes=False),
    )
    def sc_embed(idx_ref, t0, t1, t2, t3, t4, t5, t6, out_ref,
                 raw_v, idx_v, rows_v, out_v,
                 sem_g0, sem_g1, sem_i0, sem_i1, sem_o0, sem_o1):
        tabs = (t0, t1, t2, t3, t4, t5, t6)
        sem_g = (sem_g0, sem_g1)
        sem_i = (sem_i0, sem_i1)
        sem_o = (sem_o0, sem_o1)
        wid = lax.axis_index("s") * 2 + lax.axis_index("c")
        w_base = wid * n_chunks
        iota7 = lax.iota(jnp.int32, 16) * NT

        def issue_idx(ci, p):
            base = (w_base + ci) * (NT * CHUNK)
            pltpu.async_copy(idx_ref.at[pl.ds(base, NT * CHUNK)],
                             raw_v.at[p], sem_i[p])

        def wait_idx(p):
            pltpu.make_async_copy(idx_ref.at[pl.ds(0, NT * CHUNK)],
                                  raw_v.at[p], sem_i[p]).wait()

        def transpose_idx(p):
            for t in range(NT):
                for g in range(CHUNK // 16):
                    vals = plsc.load_gather(raw_v.at[p],
                                            [iota7 + (g * 16 * NT + t)])
                    idx_v[p, t, pl.ds(g * 16, 16)] = vals

        def fire_gathers(p):
            for t in range(NT):
                pltpu.async_copy(tabs[t].at[idx_v.at[p, t]],
                                 rows_v.at[p, t], sem_g[p])

        def wait_gathers(p):
            for t in range(NT):
                pltpu.make_async_copy(tabs[t].at[pl.ds(0, CHUNK)],
                                      rows_v.at[p, t], sem_g[p]).wait()

        def accumulate(p):
            def acc_body(c, carry):
                for j in range(D // 16):
                    sl = pl.ds(j * 16, 16)
                    v = rows_v[p, 0, c, sl]
                    for t in range(1, NT):
                        v = v + rows_v[p, t, c, sl]
                    out_v[p, c, sl] = v
                return carry

            lax.fori_loop(0, CHUNK, acc_body, 0, unroll=2)

        def issue_out(ci, p):
            base = (w_base + ci) * CHUNK
            pltpu.async_copy(out_v.at[p], out_ref.at[pl.ds(base, CHUNK)],
                             sem_o[p])

        def wait_out(p):
            pltpu.make_async_copy(out_v.at[p], out_ref.at[pl.ds(0, CHUNK)],
                                  sem_o[p]).wait()

        def step(ci, p, do_next, do_idx2, do_owait):
            if do_next:
                wait_idx(1 - p)
                transpose_idx(1 - p)
                fire_gathers(1 - p)
            wait_gathers(p)
            if do_idx2:
                issue_idx(ci + 2, p)
            if do_owait:
                wait_out(p)
            accumulate(p)
            issue_out(ci, p)

        issue_idx(0, 0)
        wait_idx(0)
        transpose_idx(0)
        fire_gathers(0)
        issue_idx(1, 1)

        step(0, 0, True, True, False)
        step(1, 1, True, True, False)

        def pair_body(i, carry):
            ci = 2 * i
            step(ci, 0, True, True, True)
            step(ci + 1, 1, True, True, True)
            return carry

        lax.fori_loop(1, n_chunks // 2 - 1, pair_body, 0)

        step(n_chunks - 2, 0, True, False, True)
        step(n_chunks - 1, 1, False, False, True)
        wait_out(0)
        wait_out(1)

    return sc_embed(idx_blk, *tables)


def kernel(boxes, input_boxes_counts, w_embed, h_embed, cx_embed, cy_embed,
           xskew_embed, yskew_embed, label_embed):
    del input_boxes_counts
    B, L, _ = boxes.shape
    n = B * L
    tables = (cx_embed, cy_embed, w_embed, h_embed,
              xskew_embed, yskew_embed, label_embed)
    h = B // 2
    outs = []
    for s in range(2):
        idx_flat = boxes[s * h:(s + 1) * h].reshape(h * L * NT)
        o = _sc_embed_call((idx_flat,) + tables, h * L)
        outs.append(o.reshape(h, L, D))
    return jnp.concatenate(outs, axis=0)


# speedup vs baseline: 1.0446x; 1.0446x over previous
import functools

import jax
import jax.numpy as jnp
from jax import lax
from jax.experimental import pallas as pl
from jax.experimental.pallas import tpu as pltpu
from jax.experimental.pallas import tpu_sc as plsc

D = 64
NT = 7
N_WORKERS = 32
CHUNK = 80


@functools.partial(jax.jit, static_argnums=(1,))
def _sc_embed_call(args, n):
    idx_flat, tables = args[0], args[1:]
    n_per_w = n // N_WORKERS
    n_chunks = n_per_w // CHUNK
    assert n_chunks % 2 == 0 and n_chunks >= 6
    mesh = plsc.VectorSubcoreMesh(core_axis_name="c", subcore_axis_name="s")

    @functools.partial(
        pl.kernel,
        mesh=mesh,
        out_type=jax.ShapeDtypeStruct((n, D), jnp.float32),
        scratch_types=[
            pltpu.VMEM((2, NT * CHUNK), jnp.int32),
            pltpu.VMEM((2, NT, CHUNK), jnp.int32),
            pltpu.VMEM((2, NT, CHUNK, D), jnp.float32),
            pltpu.VMEM((2, CHUNK, D), jnp.float32),
            pltpu.SemaphoreType.DMA,
            pltpu.SemaphoreType.DMA,
            pltpu.SemaphoreType.DMA,
            pltpu.SemaphoreType.DMA,
            pltpu.SemaphoreType.DMA,
            pltpu.SemaphoreType.DMA,
        ],
        compiler_params=pltpu.CompilerParams(use_tc_tiling_on_sc=False,
                                             needs_layout_passes=False),
    )
    def sc_embed(idx_ref, t0, t1, t2, t3, t4, t5, t6, out_ref,
                 raw_v, idx_v, rows_v, out_v,
                 sem_g0, sem_g1, sem_i0, sem_i1, sem_o0, sem_o1):
        tabs = (t0, t1, t2, t3, t4, t5, t6)
        sem_g = (sem_g0, sem_g1)
        sem_i = (sem_i0, sem_i1)
        sem_o = (sem_o0, sem_o1)
        wid = lax.axis_index("s") * 2 + lax.axis_index("c")
        w_base = wid * n_chunks
        iota7 = lax.iota(jnp.int32, 16) * NT

        def issue_idx(ci, p):
            base = (w_base + ci) * (NT * CHUNK)
            pltpu.async_copy(idx_ref.at[pl.ds(base, NT * CHUNK)],
                             raw_v.at[p], sem_i[p])

        def wait_idx(p):
            pltpu.make_async_copy(idx_ref.at[pl.ds(0, NT * CHUNK)],
                                  raw_v.at[p], sem_i[p]).wait()

        def transpose_idx(p):
            for t in range(NT):
                for g in range(CHUNK // 16):
                    vals = plsc.load_gather(raw_v.at[p],
                                            [iota7 + (g * 16 * NT + t)])
                    idx_v[p, t, pl.ds(g * 16, 16)] = vals

        def fire_gathers(p):
            for t in range(NT):
                pltpu.async_copy(tabs[t].at[idx_v.at[p, t]],
                                 rows_v.at[p, t], sem_g[p])

        def wait_gathers(p):
            for t in range(NT):
                pltpu.make_async_copy(tabs[t].at[pl.ds(0, CHUNK)],
                                      rows_v.at[p, t], sem_g[p]).wait()

        def accumulate(p):
            def acc_body(c, carry):
                for j in range(D // 16):
                    sl = pl.ds(j * 16, 16)
                    v = rows_v[p, 0, c, sl]
                    for t in range(1, NT):
                        v = v + rows_v[p, t, c, sl]
                    out_v[p, c, sl] = v
                return carry

            lax.fori_loop(0, CHUNK, acc_body, 0, unroll=2)

        def issue_out(ci, p):
            base = (w_base + ci) * CHUNK
            pltpu.async_copy(out_v.at[p], out_ref.at[pl.ds(base, CHUNK)],
                             sem_o[p])

        def wait_out(p):
            pltpu.make_async_copy(out_v.at[p], out_ref.at[pl.ds(0, CHUNK)],
                                  sem_o[p]).wait()

        def step(ci, p, do_next, do_idx2, do_owait):
            if do_next:
                wait_idx(1 - p)
                transpose_idx(1 - p)
                fire_gathers(1 - p)
            wait_gathers(p)
            if do_idx2:
                issue_idx(ci + 2, p)
            if do_owait:
                wait_out(p)
            accumulate(p)
            issue_out(ci, p)

        issue_idx(0, 0)
        wait_idx(0)
        transpose_idx(0)
        fire_gathers(0)
        issue_idx(1, 1)

        step(0, 0, True, True, False)
        step(1, 1, True, True, False)

        def pair_body(i, carry):
            ci = 2 * i
            step(ci, 0, True, True, True)
            step(ci + 1, 1, True, True, True)
            return carry

        lax.fori_loop(1, n_chunks // 2 - 1, pair_body, 0)

        step(n_chunks - 2, 0, True, False, True)
        step(n_chunks - 1, 1, False, False, True)
        wait_out(0)
        wait_out(1)

    return sc_embed(idx_flat, *tables)


def kernel(boxes, input_boxes_counts, w_embed, h_embed, cx_embed, cy_embed,
           xskew_embed, yskew_embed, label_embed):
    del input_boxes_counts
    B, L, _ = boxes.shape
    n = B * L
    idx_flat = boxes.reshape(n * NT)
    tables = (cx_embed, cy_embed, w_embed, h_embed,
              xskew_embed, yskew_embed, label_embed)
    out = _sc_embed_call((idx_flat,) + tables, n)
    return out.reshape(B, L, D)


# speedup vs baseline: 1.1024x; 1.0553x over previous
import functools

import jax
import jax.numpy as jnp
from jax import lax
from jax.experimental import pallas as pl
from jax.experimental.pallas import tpu as pltpu
from jax.experimental.pallas import tpu_sc as plsc

D = 64
NT = 7
N_WORKERS = 32
CHUNK = 80


@functools.partial(jax.jit, static_argnums=(1, 2))
def _sc_unpack_call(boxes, b_rows, l_len):
    n = b_rows * l_len
    bpw = b_rows // N_WORKERS
    bat = 16
    nbatch = bpw // bat
    row = l_len * NT
    reg = bat * row
    mesh = plsc.VectorSubcoreMesh(core_axis_name="c", subcore_axis_name="s")

    @functools.partial(
        pl.kernel,
        mesh=mesh,
        out_type=jax.ShapeDtypeStruct((n * NT,), jnp.int32),
        scratch_types=[
            pltpu.VMEM((2, l_len, NT), jnp.int32),
            pltpu.VMEM((reg,), jnp.int32),
            pltpu.SemaphoreType.DMA,
            pltpu.SemaphoreType.DMA,
        ],
        compiler_params=pltpu.CompilerParams(use_tc_tiling_on_sc=True,
                                             needs_layout_passes=False),
    )
    def unpack(boxes_ref, out_ref, ibuf, tbuf, sem_i0, sem_i1):
        wid = lax.axis_index("s") * 2 + lax.axis_index("c")
        iota = lax.iota(jnp.int32, 16)
        sem_i = (sem_i0, sem_i1)
        b_lo = wid * bpw
        b_hi = b_lo + bpw
        ngrp = (row + 15) // 16

        def issue_in(b, p):
            pltpu.async_copy(boxes_ref.at[b], ibuf.at[p], sem_i[p])

        def wait_in(p):
            pltpu.make_async_copy(boxes_ref.at[0], ibuf.at[p],
                                  sem_i[p]).wait()

        issue_in(b_lo, 0)
        issue_in(b_lo + 1, 1)

        def batch_body(g, carry):
            b0 = b_lo + g * bat

            def pair_body(j, carry2):
                for kk in range(2):
                    k = 2 * j + kk
                    b = b0 + k
                    wait_in(kk)

                    @pl.when(b + 2 < b_hi)
                    def _():
                        issue_in(b + 2, kk)

                    for gg in range(ngrp):
                        o0 = min(gg * 16, row - 16)
                        o = o0 + iota
                        c = o // NT
                        t = o - c * NT
                        vals = plsc.load_gather(ibuf.at[kk], [c, t])
                        tbuf[pl.ds(k * row + o0, 16)] = vals
                return carry2

            lax.fori_loop(0, bat // 2, pair_body, 0)
            dst0 = (wid * nbatch + g) * reg
            pltpu.sync_copy(tbuf, out_ref.at[pl.ds(dst0, reg)])
            return carry

        lax.fori_loop(0, nbatch, batch_body, 0)

    return unpack(boxes)


@functools.partial(jax.jit, static_argnums=(1,))
def _sc_embed_call(args, n):
    idx_flat, tables = args[0], args[1:]
    n_per_w = n // N_WORKERS
    n_chunks = n_per_w // CHUNK
    assert n_chunks % 2 == 0 and n_chunks >= 6
    mesh = plsc.VectorSubcoreMesh(core_axis_name="c", subcore_axis_name="s")

    @functools.partial(
        pl.kernel,
        mesh=mesh,
        out_type=jax.ShapeDtypeStruct((n, D), jnp.float32),
        scratch_types=[
            pltpu.VMEM((2, NT * CHUNK), jnp.int32),
            pltpu.VMEM((2, NT, CHUNK), jnp.int32),
            pltpu.VMEM((2, NT, CHUNK, D), jnp.float32),
            pltpu.VMEM((2, CHUNK, D), jnp.float32),
            pltpu.SemaphoreType.DMA,
            pltpu.SemaphoreType.DMA,
            pltpu.SemaphoreType.DMA,
            pltpu.SemaphoreType.DMA,
            pltpu.SemaphoreType.DMA,
            pltpu.SemaphoreType.DMA,
        ],
        compiler_params=pltpu.CompilerParams(use_tc_tiling_on_sc=False,
                                             needs_layout_passes=False),
    )
    def sc_embed(idx_ref, t0, t1, t2, t3, t4, t5, t6, out_ref,
                 raw_v, idx_v, rows_v, out_v,
                 sem_g0, sem_g1, sem_i0, sem_i1, sem_o0, sem_o1):
        tabs = (t0, t1, t2, t3, t4, t5, t6)
        sem_g = (sem_g0, sem_g1)
        sem_i = (sem_i0, sem_i1)
        sem_o = (sem_o0, sem_o1)
        wid = lax.axis_index("s") * 2 + lax.axis_index("c")
        w_base = wid * n_chunks
        iota7 = lax.iota(jnp.int32, 16) * NT

        def issue_idx(ci, p):
            base = (w_base + ci) * (NT * CHUNK)
            pltpu.async_copy(idx_ref.at[pl.ds(base, NT * CHUNK)],
                             raw_v.at[p], sem_i[p])

        def wait_idx(p):
            pltpu.make_async_copy(idx_ref.at[pl.ds(0, NT * CHUNK)],
                                  raw_v.at[p], sem_i[p]).wait()

        def transpose_idx(p):
            for t in range(NT):
                for g in range(CHUNK // 16):
                    vals = plsc.load_gather(raw_v.at[p],
                                            [iota7 + (g * 16 * NT + t)])
                    idx_v[p, t, pl.ds(g * 16, 16)] = vals

        def fire_gathers(p):
            for t in range(NT):
                pltpu.async_copy(tabs[t].at[idx_v.at[p, t]],
                                 rows_v.at[p, t], sem_g[p])

        def wait_gathers(p):
            for t in range(NT):
                pltpu.make_async_copy(tabs[t].at[pl.ds(0, CHUNK)],
                                      rows_v.at[p, t], sem_g[p]).wait()

        def accumulate(p):
            def acc_body(c, carry):
                for j in range(D // 16):
                    sl = pl.ds(j * 16, 16)
                    v = rows_v[p, 0, c, sl]
                    for t in range(1, NT):
                        v = v + rows_v[p, t, c, sl]
                    out_v[p, c, sl] = v
                return carry

            lax.fori_loop(0, CHUNK, acc_body, 0, unroll=2)

        def issue_out(ci, p):
            base = (w_base + ci) * CHUNK
            pltpu.async_copy(out_v.at[p], out_ref.at[pl.ds(base, CHUNK)],
                             sem_o[p])

        def wait_out(p):
            pltpu.make_async_copy(out_v.at[p], out_ref.at[pl.ds(0, CHUNK)],
                                  sem_o[p]).wait()

        def step(ci, p, do_next, do_idx2, do_owait):
            if do_next:
                wait_idx(1 - p)
                transpose_idx(1 - p)
                fire_gathers(1 - p)
            wait_gathers(p)
            if do_idx2:
                issue_idx(ci + 2, p)
            if do_owait:
                wait_out(p)
            accumulate(p)
            issue_out(ci, p)

        issue_idx(0, 0)
        wait_idx(0)
        transpose_idx(0)
        fire_gathers(0)
        issue_idx(1, 1)

        step(0, 0, True, True, False)
        step(1, 1, True, True, False)

        def pair_body(i, carry):
            ci = 2 * i
            step(ci, 0, True, True, True)
            step(ci + 1, 1, True, True, True)
            return carry

        lax.fori_loop(1, n_chunks // 2 - 1, pair_body, 0)

        step(n_chunks - 2, 0, True, False, True)
        step(n_chunks - 1, 1, False, False, True)
        wait_out(0)
        wait_out(1)

    return sc_embed(idx_flat, *tables)


def kernel(boxes, input_boxes_counts, w_embed, h_embed, cx_embed, cy_embed,
           xskew_embed, yskew_embed, label_embed):
    del input_boxes_counts
    B, L, _ = boxes.shape
    n = B * L
    idx_flat = _sc_unpack_call(boxes, B, L)
    tables = (cx_embed, cy_embed, w_embed, h_embed,
              xskew_embed, yskew_embed, label_embed)
    out = _sc_embed_call((idx_flat,) + tables, n)
    return out.reshape(B, L, D)
